# Initial kernel scaffold; baseline (speedup 1.0000x reference)
import jax
import jax.numpy as jnp
from jax.experimental import pallas as pl


def kernel(x, edge_index, batch, W1, b1, W2, b2):
    raise NotImplementedError("write your pallas kernel here")


# speedup vs baseline: 22.3782x; 22.3782x over previous
import functools

import jax
import jax.numpy as jnp
from jax import lax
from jax.experimental import pallas as pl
from jax.experimental.pallas import tpu as pltpu
from jax.experimental.pallas import tpu_sc as plsc

N_NODES = 10000
NUM_EDGES = 320000
NUM_GRAPHS = 64
NCORE = 2
NSUB = 16
NW = NCORE * NSUB
CHUNK = 128
K_CHUNKS = 79
CAP = NW * K_CHUNKS * CHUNK
N_PAD = 10112
ROWS_PER_TILE = N_PAD // NSUB
JUNK_ROW = 10100
BLK = 1000


def _mesh():
    return plsc.VectorSubcoreMesh(core_axis_name="c", subcore_axis_name="s")


def _deg_call(dst_p, ones_rows, zeros16):
    @functools.partial(
        pl.kernel,
        mesh=_mesh(),
        out_type=jax.ShapeDtypeStruct((NCORE, N_PAD, 16), jnp.float32),
        scratch_types=[
            pltpu.VMEM((K_CHUNKS, CHUNK), jnp.int32),
            pltpu.VMEM((CHUNK, 16), jnp.float32),
            pltpu.VMEM_SHARED((N_PAD, 16), jnp.float32),
        ],
        compiler_params=pltpu.CompilerParams(use_tc_tiling_on_sc=False),
    )
    def deg_k(dst_hbm, ones_hbm, zeros_hbm, out_hbm, idx_v, ones_v, bins_sh):
        c = lax.axis_index("c")
        s = lax.axis_index("s")
        wid = c * NSUB + s
        r0 = pl.multiple_of(s * ROWS_PER_TILE, 8)
        pltpu.sync_copy(zeros_hbm.at[pl.ds(r0, ROWS_PER_TILE)],
                        bins_sh.at[pl.ds(r0, ROWS_PER_TILE)])
        pltpu.sync_copy(ones_hbm, ones_v)
        pltpu.sync_copy(dst_hbm.at[wid], idx_v)
        plsc.subcore_barrier()

        def body(j, carry):
            pltpu.sync_copy(ones_v, bins_sh.at[idx_v.at[j]], add=True)
            return carry

        lax.fori_loop(0, K_CHUNKS, body, 0)
        plsc.subcore_barrier()
        pltpu.sync_copy(bins_sh.at[pl.ds(r0, ROWS_PER_TILE)],
                        out_hbm.at[c, pl.ds(r0, ROWS_PER_TILE)])

    return deg_k(dst_p, ones_rows, zeros16)


def _prop_call(u, src_p, dst_p, zeros, d):
    @functools.partial(
        pl.kernel,
        mesh=_mesh(),
        out_type=jax.ShapeDtypeStruct((NCORE, N_PAD, d), jnp.float32),
        scratch_types=[
            pltpu.VMEM((K_CHUNKS, CHUNK), jnp.int32),
            pltpu.VMEM((K_CHUNKS, CHUNK), jnp.int32),
            pltpu.VMEM((CHUNK, d), jnp.float32),
            pltpu.VMEM_SHARED((N_PAD, d), jnp.float32),
            pltpu.SemaphoreType.DMA,
        ],
        compiler_params=pltpu.CompilerParams(use_tc_tiling_on_sc=False),
    )
    def prop_k(u_hbm, src_hbm, dst_hbm, zeros_hbm, out_hbm,
               src_v, dst_v, rows_v, acc_sh, sem):
        c = lax.axis_index("c")
        s = lax.axis_index("s")
        wid = c * NSUB + s
        r0 = pl.multiple_of(s * ROWS_PER_TILE, 8)
        pltpu.sync_copy(zeros_hbm.at[pl.ds(r0, ROWS_PER_TILE)],
                        acc_sh.at[pl.ds(r0, ROWS_PER_TILE)])
        pltpu.sync_copy(src_hbm.at[wid], src_v)
        pltpu.sync_copy(dst_hbm.at[wid], dst_v)
        plsc.subcore_barrier()

        def body(j, carry):
            pltpu.async_copy(u_hbm.at[src_v.at[j]], rows_v, sem).wait()
            pltpu.sync_copy(rows_v, acc_sh.at[dst_v.at[j]], add=True)
            return carry

        lax.fori_loop(0, K_CHUNKS, body, 0)
        plsc.subcore_barrier()
        pltpu.sync_copy(acc_sh.at[pl.ds(r0, ROWS_PER_TILE)],
                        out_hbm.at[c, pl.ds(r0, ROWS_PER_TILE)])

    return prop_k(u, src_p, dst_p, zeros)


def _tc1_call(bins, x, W1):
    def body(bins_ref, x_ref, w_ref, u_ref, dis_ref):
        deg = bins_ref[0][:, 0:1] + bins_ref[1][:, 0:1] + 1.0
        dis = lax.rsqrt(deg)
        h = jnp.dot(x_ref[...], w_ref[...], preferred_element_type=jnp.float32)
        u_ref[...] = h * dis
        dis_ref[...] = dis

    return pl.pallas_call(
        body,
        grid=(N_NODES // BLK,),
        in_specs=[
            pl.BlockSpec((NCORE, BLK, 16), lambda i: (0, i, 0)),
            pl.BlockSpec((BLK, 128), lambda i: (i, 0)),
            pl.BlockSpec((128, 64), lambda i: (0, 0)),
        ],
        out_specs=[
            pl.BlockSpec((BLK, 64), lambda i: (i, 0)),
            pl.BlockSpec((BLK, 1), lambda i: (i, 0)),
        ],
        out_shape=[
            jax.ShapeDtypeStruct((N_NODES, 64), jnp.float32),
            jax.ShapeDtypeStruct((N_NODES, 1), jnp.float32),
        ],
    )(bins, x, W1)


def _tc2_call(p, u1, dis, b1, W2):
    def body(p_ref, u1_ref, dis_ref, b1_ref, w2_ref, u2_ref):
        sfull = p_ref[0] + p_ref[1] + u1_ref[...]
        o = jnp.maximum(sfull * dis_ref[...] + b1_ref[...], 0.0)
        u2_ref[...] = jnp.dot(o, w2_ref[...],
                              preferred_element_type=jnp.float32) * dis_ref[...]

    return pl.pallas_call(
        body,
        grid=(N_NODES // BLK,),
        in_specs=[
            pl.BlockSpec((NCORE, BLK, 64), lambda i: (0, i, 0)),
            pl.BlockSpec((BLK, 64), lambda i: (i, 0)),
            pl.BlockSpec((BLK, 1), lambda i: (i, 0)),
            pl.BlockSpec((1, 64), lambda i: (0, 0)),
            pl.BlockSpec((64, 32), lambda i: (0, 0)),
        ],
        out_specs=pl.BlockSpec((BLK, 32), lambda i: (i, 0)),
        out_shape=jax.ShapeDtypeStruct((N_NODES, 32), jnp.float32),
    )(p, u1, dis, b1, W2)


def _tc3_call(p, u2, dis, b2, batch_r):
    def body(p_ref, u2_ref, dis_ref, b2_ref, bt_ref, out_ref):
        h2 = (p_ref[0] + p_ref[1] + u2_ref[...]) * dis_ref[...] + b2_ref[...]
        bt = bt_ref[0]
        oh = (lax.broadcasted_iota(jnp.int32, (NUM_GRAPHS, BLK), 0)
              == bt).astype(jnp.float32)
        acc = jnp.dot(oh, h2, preferred_element_type=jnp.float32)
        i = pl.program_id(0)

        @pl.when(i == 0)
        def _():
            out_ref[...] = acc

        @pl.when(i != 0)
        def _():
            out_ref[...] += acc

    return pl.pallas_call(
        body,
        grid=(N_NODES // BLK,),
        in_specs=[
            pl.BlockSpec((NCORE, BLK, 32), lambda i: (0, i, 0)),
            pl.BlockSpec((BLK, 32), lambda i: (i, 0)),
            pl.BlockSpec((BLK, 1), lambda i: (i, 0)),
            pl.BlockSpec((1, 32), lambda i: (0, 0)),
            pl.BlockSpec((1, 1, BLK), lambda i: (i, 0, 0)),
        ],
        out_specs=pl.BlockSpec((NUM_GRAPHS, 32), lambda i: (0, 0)),
        out_shape=jax.ShapeDtypeStruct((NUM_GRAPHS, 32), jnp.float32),
    )(p, u2, dis, b2, batch_r)


def kernel(x, edge_index, batch, W1, b1, W2, b2):
    src = edge_index[0].astype(jnp.int32)
    dst = edge_index[1].astype(jnp.int32)
    pad = CAP - NUM_EDGES
    src_p = jnp.concatenate(
        [src, jnp.zeros((pad,), jnp.int32)]).reshape(NW, K_CHUNKS, CHUNK)
    dst_p = jnp.concatenate(
        [dst, jnp.full((pad,), JUNK_ROW, jnp.int32)]).reshape(NW, K_CHUNKS, CHUNK)
    ones_rows = jnp.ones((CHUNK, 16), jnp.float32)
    z16 = jnp.zeros((N_PAD, 16), jnp.float32)
    z64 = jnp.zeros((N_PAD, 64), jnp.float32)
    z32 = jnp.zeros((N_PAD, 32), jnp.float32)

    bins = _deg_call(dst_p, ones_rows, z16)
    u1, dis = _tc1_call(bins[:, :N_NODES, :], x, W1)
    p1 = _prop_call(u1, src_p, dst_p, z64, 64)
    u2 = _tc2_call(p1[:, :N_NODES, :], u1, dis, b1.reshape(1, 64), W2)
    p2 = _prop_call(u2, src_p, dst_p, z32, 32)
    out = _tc3_call(p2[:, :N_NODES, :], u2, dis, b2.reshape(1, 32),
                    batch.astype(jnp.int32).reshape(N_NODES // BLK, 1, BLK))
    return out
